# baseline (speedup 1.0000x reference)
import functools

import jax
import jax.numpy as jnp
from jax import lax
from jax.experimental import pallas as pl
from jax.experimental.pallas import tpu as pltpu
from jax.experimental.pallas import tpu_sc as plsc

N = 16384
K = 8192
D = 64
BI = 512
BL = 2048
COMMIT = 0.25


def _argmin_onehot_body(x_ref, c_ref, xsq_ref, csq_ref,
                        idx_ref, oh_ref, perp_ref, cnt_ref):
    i = pl.program_id(0)
    x = x_ref[...]
    c = c_ref[...]
    m = lax.dot_general(x.astype(jnp.bfloat16), c.astype(jnp.bfloat16),
                        (((1,), (1,)), ((), ())),
                        preferred_element_type=jnp.float32)
    d = (xsq_ref[...] + csq_ref[...]) - 2.0 * m
    H = K // 2
    d1 = d[:, :H]
    d2 = d[:, H:]
    colh = lax.broadcasted_iota(jnp.int32, (BI, H), 1)
    v1 = jnp.min(d1, axis=1, keepdims=True)
    i1 = jnp.min(jnp.where(d1 == v1, colh, K), axis=1, keepdims=True)
    v2 = jnp.min(d2, axis=1, keepdims=True)
    i2 = jnp.min(jnp.where(d2 == v2, colh, K), axis=1, keepdims=True) + H
    v1b = v1.astype(jnp.bfloat16).astype(jnp.float32)
    idx = jnp.where(v2 < v1b, i2, i1)
    idx_ref[...] = idx
    one = jnp.float32(1.0)
    zero = jnp.float32(0.0)
    oh1 = jnp.where(colh == idx, one, zero)
    oh2 = jnp.where(colh == idx - H, one, zero)
    oh_ref[:, :H] = oh1
    oh_ref[:, H:] = oh2
    cs1 = jnp.sum(oh1, axis=0, keepdims=True)
    cs2 = jnp.sum(oh2, axis=0, keepdims=True)

    @pl.when(i == 0)
    def _():
        cnt_ref[:, :H] = cs1
        cnt_ref[:, H:] = cs2

    @pl.when(i > 0)
    def _():
        cnt_ref[:, :H] = cnt_ref[:, :H] + cs1
        cnt_ref[:, H:] = cnt_ref[:, H:] + cs2

    @pl.when(i == N // BI - 1)
    def _():
        p = cnt_ref[...] * jnp.float32(1.0 / N)
        ent = jnp.sum(p * jnp.log(p + jnp.float32(1e-10)), keepdims=True)
        perp_ref[...] = jnp.exp(-ent).reshape(1, 1)


_argmin_onehot = pl.pallas_call(
    _argmin_onehot_body,
    grid=(N // BI,),
    in_specs=[
        pl.BlockSpec((BI, D), lambda i: (i, 0)),
        pl.BlockSpec((K, D), lambda i: (0, 0)),
        pl.BlockSpec((BI, 1), lambda i: (i, 0)),
        pl.BlockSpec((1, K), lambda i: (0, 0)),
    ],
    out_specs=[
        pl.BlockSpec((BI, 1), lambda i: (i, 0)),
        pl.BlockSpec((BI, K), lambda i: (i, 0)),
        pl.BlockSpec((1, 1), lambda i: (0, 0)),
    ],
    out_shape=[
        jax.ShapeDtypeStruct((N, 1), jnp.int32),
        jax.ShapeDtypeStruct((N, K), jnp.float32),
        jax.ShapeDtypeStruct((1, 1), jnp.float32),
    ],
    scratch_shapes=[pltpu.VMEM((1, K), jnp.float32)],
)


_info = plsc.get_sparse_core_info()
_NC, _NS = _info.num_cores, _info.num_subcores
_NW = _NC * _NS
_BPW = N // _NW


_DP = 128
_CH = 128


@functools.partial(
    pl.kernel,
    mesh=plsc.VectorSubcoreMesh(core_axis_name="c", subcore_axis_name="s"),
    out_type=jax.ShapeDtypeStruct((N, _DP), jnp.float32),
    scratch_types=[
        pltpu.VMEM((_BPW,), jnp.int32),
        pltpu.VMEM((_BPW, _DP), jnp.float32),
        pltpu.SemaphoreType.DMA,
    ],
)
def _sc_gather(idx_hbm, table_hbm, out_hbm, idx_v, rows_v, sem):
    wid = lax.axis_index("s") * _NC + lax.axis_index("c")
    base = wid * _BPW
    pltpu.sync_copy(idx_hbm.at[pl.ds(base, _BPW)], idx_v)
    copies = [
        pltpu.async_copy(
            table_hbm.at[idx_v.at[pl.ds(j * _CH, _CH)]],
            rows_v.at[pl.ds(j * _CH, _CH)], sem)
        for j in range(_BPW // _CH)
    ]
    for cp in copies:
        cp.wait()
    pltpu.sync_copy(rows_v, out_hbm.at[pl.ds(base, _BPW)])


def _loss_body(x_ref, q_ref, qst_ref, loss_ref, acc_ref):
    i = pl.program_id(0)
    x = x_ref[...]
    q = q_ref[...]
    dqx = q - x
    qst_ref[...] = x + dqx
    s = jnp.sum(dqx * dqx)

    @pl.when(i == 0)
    def _():
        acc_ref[0] = s

    @pl.when(i > 0)
    def _():
        acc_ref[0] = acc_ref[0] + s

    @pl.when(i == N // BL - 1)
    def _():
        mse = acc_ref[0] * jnp.float32(1.0 / (N * D))
        loss_ref[...] = jnp.full((1, 1), mse + jnp.float32(COMMIT) * mse,
                                 dtype=jnp.float32)


_loss = pl.pallas_call(
    _loss_body,
    grid=(N // BL,),
    in_specs=[
        pl.BlockSpec((BL, D), lambda i: (i, 0)),
        pl.BlockSpec((BL, D), lambda i: (i, 0)),
    ],
    out_specs=[
        pl.BlockSpec((BL, D), lambda i: (i, 0)),
        pl.BlockSpec((1, 1), lambda i: (0, 0)),
    ],
    out_shape=[
        jax.ShapeDtypeStruct((N, D), jnp.float32),
        jax.ShapeDtypeStruct((1, 1), jnp.float32),
    ],
    scratch_shapes=[pltpu.SMEM((1,), jnp.float32)],
)


def kernel(inputs, codebook):
    xsq = jnp.sum(inputs ** 2, axis=1, keepdims=True)
    csq = jnp.sum(codebook ** 2, axis=1).reshape(1, K)
    idx2d, encodings, perp = _argmin_onehot(inputs, codebook, xsq, csq)
    encoding_indices = idx2d.reshape(N)
    table_p = jnp.pad(codebook, ((0, 0), (0, _DP - D)))
    quantized = _sc_gather(encoding_indices, table_p)[:, :D]
    quantized_st, loss = _loss(inputs, quantized)
    return (quantized_st, jnp.reshape(perp, ()), encodings,
            encoding_indices, jnp.reshape(loss, ()))


# speedup vs baseline: 1.0055x; 1.0055x over previous
import functools

import jax
import jax.numpy as jnp
from jax import lax
from jax.experimental import pallas as pl
from jax.experimental.pallas import tpu as pltpu
from jax.experimental.pallas import tpu_sc as plsc

N = 16384
K = 8192
D = 64
BI = 256
BL = 2048
COMMIT = 0.25


def _argmin_onehot_body(x_ref, c_ref, xsq_ref, csq_ref,
                        idx_ref, oh_ref, perp_ref, cnt_ref):
    i = pl.program_id(0)
    x = x_ref[...]
    c = c_ref[...]
    m = lax.dot_general(x.astype(jnp.bfloat16), c.astype(jnp.bfloat16),
                        (((1,), (1,)), ((), ())),
                        preferred_element_type=jnp.float32)
    d = (xsq_ref[...] + csq_ref[...]) - 2.0 * m
    H = K // 2
    d1 = d[:, :H]
    d2 = d[:, H:]
    colh = lax.broadcasted_iota(jnp.int32, (BI, H), 1)
    v1 = jnp.min(d1, axis=1, keepdims=True)
    i1 = jnp.min(jnp.where(d1 == v1, colh, K), axis=1, keepdims=True)
    v2 = jnp.min(d2, axis=1, keepdims=True)
    i2 = jnp.min(jnp.where(d2 == v2, colh, K), axis=1, keepdims=True) + H
    v1b = v1.astype(jnp.bfloat16).astype(jnp.float32)
    idx = jnp.where(v2 < v1b, i2, i1)
    idx_ref[...] = idx
    one = jnp.float32(1.0)
    zero = jnp.float32(0.0)
    oh1 = jnp.where(colh == idx, one, zero)
    oh2 = jnp.where(colh == idx - H, one, zero)
    oh_ref[:, :H] = oh1
    oh_ref[:, H:] = oh2
    cs1 = jnp.sum(oh1, axis=0, keepdims=True)
    cs2 = jnp.sum(oh2, axis=0, keepdims=True)

    @pl.when(i == 0)
    def _():
        cnt_ref[:, :H] = cs1
        cnt_ref[:, H:] = cs2

    @pl.when(i > 0)
    def _():
        cnt_ref[:, :H] = cnt_ref[:, :H] + cs1
        cnt_ref[:, H:] = cnt_ref[:, H:] + cs2

    @pl.when(i == N // BI - 1)
    def _():
        p = cnt_ref[...] * jnp.float32(1.0 / N)
        ent = jnp.sum(p * jnp.log(p + jnp.float32(1e-10)), keepdims=True)
        perp_ref[...] = jnp.exp(-ent).reshape(1, 1)


_argmin_onehot = pl.pallas_call(
    _argmin_onehot_body,
    grid=(N // BI,),
    in_specs=[
        pl.BlockSpec((BI, D), lambda i: (i, 0)),
        pl.BlockSpec((K, D), lambda i: (0, 0)),
        pl.BlockSpec((BI, 1), lambda i: (i, 0)),
        pl.BlockSpec((1, K), lambda i: (0, 0)),
    ],
    out_specs=[
        pl.BlockSpec((BI, 1), lambda i: (i, 0)),
        pl.BlockSpec((BI, K), lambda i: (i, 0)),
        pl.BlockSpec((1, 1), lambda i: (0, 0)),
    ],
    out_shape=[
        jax.ShapeDtypeStruct((N, 1), jnp.int32),
        jax.ShapeDtypeStruct((N, K), jnp.float32),
        jax.ShapeDtypeStruct((1, 1), jnp.float32),
    ],
    scratch_shapes=[pltpu.VMEM((1, K), jnp.float32)],
)


_info = plsc.get_sparse_core_info()
_NC, _NS = _info.num_cores, _info.num_subcores
_NW = _NC * _NS
_BPW = N // _NW


_DP = 128
_CH = 128


@functools.partial(
    pl.kernel,
    mesh=plsc.VectorSubcoreMesh(core_axis_name="c", subcore_axis_name="s"),
    out_type=jax.ShapeDtypeStruct((N, _DP), jnp.float32),
    scratch_types=[
        pltpu.VMEM((_BPW,), jnp.int32),
        pltpu.VMEM((_BPW, _DP), jnp.float32),
        pltpu.SemaphoreType.DMA,
    ],
)
def _sc_gather(idx_hbm, table_hbm, out_hbm, idx_v, rows_v, sem):
    wid = lax.axis_index("s") * _NC + lax.axis_index("c")
    base = wid * _BPW
    pltpu.sync_copy(idx_hbm.at[pl.ds(base, _BPW)], idx_v)
    copies = [
        pltpu.async_copy(
            table_hbm.at[idx_v.at[pl.ds(j * _CH, _CH)]],
            rows_v.at[pl.ds(j * _CH, _CH)], sem)
        for j in range(_BPW // _CH)
    ]
    for cp in copies:
        cp.wait()
    pltpu.sync_copy(rows_v, out_hbm.at[pl.ds(base, _BPW)])


def _loss_body(x_ref, q_ref, qst_ref, loss_ref, acc_ref):
    i = pl.program_id(0)
    x = x_ref[...]
    q = q_ref[...]
    dqx = q - x
    qst_ref[...] = x + dqx
    s = jnp.sum(dqx * dqx)

    @pl.when(i == 0)
    def _():
        acc_ref[0] = s

    @pl.when(i > 0)
    def _():
        acc_ref[0] = acc_ref[0] + s

    @pl.when(i == N // BL - 1)
    def _():
        mse = acc_ref[0] * jnp.float32(1.0 / (N * D))
        loss_ref[...] = jnp.full((1, 1), mse + jnp.float32(COMMIT) * mse,
                                 dtype=jnp.float32)


_loss = pl.pallas_call(
    _loss_body,
    grid=(N // BL,),
    in_specs=[
        pl.BlockSpec((BL, D), lambda i: (i, 0)),
        pl.BlockSpec((BL, D), lambda i: (i, 0)),
    ],
    out_specs=[
        pl.BlockSpec((BL, D), lambda i: (i, 0)),
        pl.BlockSpec((1, 1), lambda i: (0, 0)),
    ],
    out_shape=[
        jax.ShapeDtypeStruct((N, D), jnp.float32),
        jax.ShapeDtypeStruct((1, 1), jnp.float32),
    ],
    scratch_shapes=[pltpu.SMEM((1,), jnp.float32)],
)


def kernel(inputs, codebook):
    xsq = jnp.sum(inputs ** 2, axis=1, keepdims=True)
    csq = jnp.sum(codebook ** 2, axis=1).reshape(1, K)
    idx2d, encodings, perp = _argmin_onehot(inputs, codebook, xsq, csq)
    encoding_indices = idx2d.reshape(N)
    table_p = jnp.pad(codebook, ((0, 0), (0, _DP - D)))
    quantized = _sc_gather(encoding_indices, table_p)[:, :D]
    quantized_st, loss = _loss(inputs, quantized)
    return (quantized_st, jnp.reshape(perp, ()), encodings,
            encoding_indices, jnp.reshape(loss, ()))


# speedup vs baseline: 1.0631x; 1.0573x over previous
import functools

import jax
import jax.numpy as jnp
from jax import lax
from jax.experimental import pallas as pl
from jax.experimental.pallas import tpu as pltpu
from jax.experimental.pallas import tpu_sc as plsc

N = 16384
K = 8192
D = 64
BI = 256
BL = 2048
COMMIT = 0.25


def _argmin_onehot_body(x_ref, c_ref, xsq_ref, csq_ref,
                        idx_ref, oh_ref, perp_ref, cnt_ref):
    i = pl.program_id(0)
    x = x_ref[...]
    c = c_ref[...]
    m = lax.dot_general(x.astype(jnp.bfloat16), c.astype(jnp.bfloat16),
                        (((1,), (1,)), ((), ())),
                        preferred_element_type=jnp.float32)
    d = (xsq_ref[...] + csq_ref[...]) - 2.0 * m
    H = K // 2
    d1 = d[:, :H]
    d2 = d[:, H:]
    colf = lax.broadcasted_iota(jnp.int32, (BI, H), 1).astype(jnp.float32)
    bigf = jnp.float32(K)
    hf = jnp.float32(H)
    v1 = jnp.min(d1, axis=1, keepdims=True)
    i1f = jnp.min(jnp.where(d1 == v1, colf, bigf), axis=1, keepdims=True)
    v2 = jnp.min(d2, axis=1, keepdims=True)
    i2f = jnp.min(jnp.where(d2 == v2, colf, bigf), axis=1, keepdims=True) + hf
    v1b = v1.astype(jnp.bfloat16).astype(jnp.float32)
    idxf = jnp.where(v2 < v1b, i2f, i1f)
    idx_ref[...] = idxf.astype(jnp.int32)
    one = jnp.float32(1.0)
    zero = jnp.float32(0.0)
    oh1 = jnp.where(colf == idxf, one, zero)
    oh2 = jnp.where(colf == idxf - hf, one, zero)
    oh_ref[:, :H] = oh1
    oh_ref[:, H:] = oh2
    cs1 = jnp.sum(oh1, axis=0, keepdims=True)
    cs2 = jnp.sum(oh2, axis=0, keepdims=True)

    @pl.when(i == 0)
    def _():
        cnt_ref[:, :H] = cs1
        cnt_ref[:, H:] = cs2

    @pl.when(i > 0)
    def _():
        cnt_ref[:, :H] = cnt_ref[:, :H] + cs1
        cnt_ref[:, H:] = cnt_ref[:, H:] + cs2

    @pl.when(i == N // BI - 1)
    def _():
        p = cnt_ref[...] * jnp.float32(1.0 / N)
        ent = jnp.sum(p * jnp.log(p + jnp.float32(1e-10)), keepdims=True)
        perp_ref[...] = jnp.exp(-ent).reshape(1, 1)


_argmin_onehot = pl.pallas_call(
    _argmin_onehot_body,
    grid=(N // BI,),
    in_specs=[
        pl.BlockSpec((BI, D), lambda i: (i, 0)),
        pl.BlockSpec((K, D), lambda i: (0, 0)),
        pl.BlockSpec((BI, 1), lambda i: (i, 0)),
        pl.BlockSpec((1, K), lambda i: (0, 0)),
    ],
    out_specs=[
        pl.BlockSpec((BI, 1), lambda i: (i, 0)),
        pl.BlockSpec((BI, K), lambda i: (i, 0)),
        pl.BlockSpec((1, 1), lambda i: (0, 0)),
    ],
    out_shape=[
        jax.ShapeDtypeStruct((N, 1), jnp.int32),
        jax.ShapeDtypeStruct((N, K), jnp.float32),
        jax.ShapeDtypeStruct((1, 1), jnp.float32),
    ],
    scratch_shapes=[pltpu.VMEM((1, K), jnp.float32)],
)


_info = plsc.get_sparse_core_info()
_NC, _NS = _info.num_cores, _info.num_subcores
_NW = _NC * _NS
_BPW = N // _NW


_DP = 128
_CH = 128


@functools.partial(
    pl.kernel,
    mesh=plsc.VectorSubcoreMesh(core_axis_name="c", subcore_axis_name="s"),
    out_type=jax.ShapeDtypeStruct((N, _DP), jnp.float32),
    scratch_types=[
        pltpu.VMEM((_BPW,), jnp.int32),
        pltpu.VMEM((_BPW, _DP), jnp.float32),
        pltpu.SemaphoreType.DMA,
    ],
)
def _sc_gather(idx_hbm, table_hbm, out_hbm, idx_v, rows_v, sem):
    wid = lax.axis_index("s") * _NC + lax.axis_index("c")
    base = wid * _BPW
    pltpu.sync_copy(idx_hbm.at[pl.ds(base, _BPW)], idx_v)
    copies = [
        pltpu.async_copy(
            table_hbm.at[idx_v.at[pl.ds(j * _CH, _CH)]],
            rows_v.at[pl.ds(j * _CH, _CH)], sem)
        for j in range(_BPW // _CH)
    ]
    for cp in copies:
        cp.wait()
    pltpu.sync_copy(rows_v, out_hbm.at[pl.ds(base, _BPW)])


def _loss_body(x_ref, q_ref, qst_ref, loss_ref, acc_ref):
    i = pl.program_id(0)
    x = x_ref[...]
    q = q_ref[...]
    dqx = q - x
    qst_ref[...] = x + dqx
    s = jnp.sum(dqx * dqx)

    @pl.when(i == 0)
    def _():
        acc_ref[0] = s

    @pl.when(i > 0)
    def _():
        acc_ref[0] = acc_ref[0] + s

    @pl.when(i == N // BL - 1)
    def _():
        mse = acc_ref[0] * jnp.float32(1.0 / (N * D))
        loss_ref[...] = jnp.full((1, 1), mse + jnp.float32(COMMIT) * mse,
                                 dtype=jnp.float32)


_loss = pl.pallas_call(
    _loss_body,
    grid=(N // BL,),
    in_specs=[
        pl.BlockSpec((BL, D), lambda i: (i, 0)),
        pl.BlockSpec((BL, D), lambda i: (i, 0)),
    ],
    out_specs=[
        pl.BlockSpec((BL, D), lambda i: (i, 0)),
        pl.BlockSpec((1, 1), lambda i: (0, 0)),
    ],
    out_shape=[
        jax.ShapeDtypeStruct((N, D), jnp.float32),
        jax.ShapeDtypeStruct((1, 1), jnp.float32),
    ],
    scratch_shapes=[pltpu.SMEM((1,), jnp.float32)],
)


def kernel(inputs, codebook):
    xsq = jnp.sum(inputs ** 2, axis=1, keepdims=True)
    csq = jnp.sum(codebook ** 2, axis=1).reshape(1, K)
    idx2d, encodings, perp = _argmin_onehot(inputs, codebook, xsq, csq)
    encoding_indices = idx2d.reshape(N)
    table_p = jnp.pad(codebook, ((0, 0), (0, _DP - D)))
    quantized = _sc_gather(encoding_indices, table_p)[:, :D]
    quantized_st, loss = _loss(inputs, quantized)
    return (quantized_st, jnp.reshape(perp, ()), encodings,
            encoding_indices, jnp.reshape(loss, ()))


# speedup vs baseline: 1.0845x; 1.0201x over previous
import functools

import jax
import jax.numpy as jnp
from jax import lax
from jax.experimental import pallas as pl
from jax.experimental.pallas import tpu as pltpu
from jax.experimental.pallas import tpu_sc as plsc

N = 16384
K = 8192
D = 64
BI = 512
BL = 2048
COMMIT = 0.25


def _argmin_onehot_body(x_ref, c_ref, xsq_ref, csq_ref,
                        idx_ref, oh_ref, perp_ref, cnt_ref):
    i = pl.program_id(0)
    x = x_ref[...]
    c = c_ref[...]
    m = lax.dot_general(x.astype(jnp.bfloat16), c.astype(jnp.bfloat16),
                        (((1,), (1,)), ((), ())),
                        preferred_element_type=jnp.float32)
    d = (xsq_ref[...] + csq_ref[...]) - 2.0 * m
    H = K // 2
    d1 = d[:, :H]
    d2 = d[:, H:]
    colf = lax.broadcasted_iota(jnp.int32, (BI, H), 1).astype(jnp.float32)
    bigf = jnp.float32(K)
    hf = jnp.float32(H)
    v1 = jnp.min(d1, axis=1, keepdims=True)
    i1f = jnp.min(jnp.where(d1 == v1, colf, bigf), axis=1, keepdims=True)
    v2 = jnp.min(d2, axis=1, keepdims=True)
    i2f = jnp.min(jnp.where(d2 == v2, colf, bigf), axis=1, keepdims=True) + hf
    v1b = v1.astype(jnp.bfloat16).astype(jnp.float32)
    idxf = jnp.where(v2 < v1b, i2f, i1f)
    idx_ref[...] = idxf.astype(jnp.int32)
    one = jnp.float32(1.0)
    zero = jnp.float32(0.0)
    oh1 = jnp.where(colf == idxf, one, zero)
    oh2 = jnp.where(colf == idxf - hf, one, zero)
    oh_ref[:, :H] = oh1
    oh_ref[:, H:] = oh2
    cs1 = jnp.sum(oh1, axis=0, keepdims=True)
    cs2 = jnp.sum(oh2, axis=0, keepdims=True)

    @pl.when(i == 0)
    def _():
        cnt_ref[:, :H] = cs1
        cnt_ref[:, H:] = cs2

    @pl.when(i > 0)
    def _():
        cnt_ref[:, :H] = cnt_ref[:, :H] + cs1
        cnt_ref[:, H:] = cnt_ref[:, H:] + cs2

    @pl.when(i == N // BI - 1)
    def _():
        p = cnt_ref[...] * jnp.float32(1.0 / N)
        ent = jnp.sum(p * jnp.log(p + jnp.float32(1e-10)), keepdims=True)
        perp_ref[...] = jnp.exp(-ent).reshape(1, 1)


_argmin_onehot = pl.pallas_call(
    _argmin_onehot_body,
    grid=(N // BI,),
    in_specs=[
        pl.BlockSpec((BI, D), lambda i: (i, 0)),
        pl.BlockSpec((K, D), lambda i: (0, 0)),
        pl.BlockSpec((BI, 1), lambda i: (i, 0)),
        pl.BlockSpec((1, K), lambda i: (0, 0)),
    ],
    out_specs=[
        pl.BlockSpec((BI, 1), lambda i: (i, 0)),
        pl.BlockSpec((BI, K), lambda i: (i, 0)),
        pl.BlockSpec((1, 1), lambda i: (0, 0)),
    ],
    out_shape=[
        jax.ShapeDtypeStruct((N, 1), jnp.int32),
        jax.ShapeDtypeStruct((N, K), jnp.float32),
        jax.ShapeDtypeStruct((1, 1), jnp.float32),
    ],
    scratch_shapes=[pltpu.VMEM((1, K), jnp.float32)],
)


_info = plsc.get_sparse_core_info()
_NC, _NS = _info.num_cores, _info.num_subcores
_NW = _NC * _NS
_BPW = N // _NW


_DP = 128
_CH = 128


@functools.partial(
    pl.kernel,
    mesh=plsc.VectorSubcoreMesh(core_axis_name="c", subcore_axis_name="s"),
    out_type=jax.ShapeDtypeStruct((N, _DP), jnp.float32),
    scratch_types=[
        pltpu.VMEM((_BPW,), jnp.int32),
        pltpu.VMEM((_BPW, _DP), jnp.float32),
        pltpu.SemaphoreType.DMA,
    ],
)
def _sc_gather(idx_hbm, table_hbm, out_hbm, idx_v, rows_v, sem):
    wid = lax.axis_index("s") * _NC + lax.axis_index("c")
    base = wid * _BPW
    pltpu.sync_copy(idx_hbm.at[pl.ds(base, _BPW)], idx_v)
    copies = [
        pltpu.async_copy(
            table_hbm.at[idx_v.at[pl.ds(j * _CH, _CH)]],
            rows_v.at[pl.ds(j * _CH, _CH)], sem)
        for j in range(_BPW // _CH)
    ]
    for cp in copies:
        cp.wait()
    pltpu.sync_copy(rows_v, out_hbm.at[pl.ds(base, _BPW)])


def _loss_body(x_ref, q_ref, qst_ref, loss_ref, acc_ref):
    i = pl.program_id(0)
    x = x_ref[...]
    q = q_ref[...]
    dqx = q - x
    qst_ref[...] = x + dqx
    s = jnp.sum(dqx * dqx)

    @pl.when(i == 0)
    def _():
        acc_ref[0] = s

    @pl.when(i > 0)
    def _():
        acc_ref[0] = acc_ref[0] + s

    @pl.when(i == N // BL - 1)
    def _():
        mse = acc_ref[0] * jnp.float32(1.0 / (N * D))
        loss_ref[...] = jnp.full((1, 1), mse + jnp.float32(COMMIT) * mse,
                                 dtype=jnp.float32)


_loss = pl.pallas_call(
    _loss_body,
    grid=(N // BL,),
    in_specs=[
        pl.BlockSpec((BL, D), lambda i: (i, 0)),
        pl.BlockSpec((BL, D), lambda i: (i, 0)),
    ],
    out_specs=[
        pl.BlockSpec((BL, D), lambda i: (i, 0)),
        pl.BlockSpec((1, 1), lambda i: (0, 0)),
    ],
    out_shape=[
        jax.ShapeDtypeStruct((N, D), jnp.float32),
        jax.ShapeDtypeStruct((1, 1), jnp.float32),
    ],
    scratch_shapes=[pltpu.SMEM((1,), jnp.float32)],
)


def kernel(inputs, codebook):
    xsq = jnp.sum(inputs ** 2, axis=1, keepdims=True)
    csq = jnp.sum(codebook ** 2, axis=1).reshape(1, K)
    idx2d, encodings, perp = _argmin_onehot(inputs, codebook, xsq, csq)
    encoding_indices = idx2d.reshape(N)
    table_p = jnp.pad(codebook, ((0, 0), (0, _DP - D)))
    quantized = _sc_gather(encoding_indices, table_p)[:, :D]
    quantized_st, loss = _loss(inputs, quantized)
    return (quantized_st, jnp.reshape(perp, ()), encodings,
            encoding_indices, jnp.reshape(loss, ()))
